# Initial kernel scaffold; baseline (speedup 1.0000x reference)
import jax
import jax.numpy as jnp
from jax.experimental import pallas as pl


def kernel(x, edge_index, W1, b1, W2, b2, W3, b3, W4, b4, Wc1, bc1, prelu_a, Wc2, bc2):
    raise NotImplementedError("write your pallas kernel here")


# speedup vs baseline: 2.7813x; 2.7813x over previous
import functools

import jax
import jax.numpy as jnp
from jax import lax
from jax.experimental import pallas as pl
from jax.experimental.pallas import tpu as pltpu
from jax.experimental.pallas import tpu_sc as plsc

N = 10000
E = 320000
D = 128
CH = 128
EP = 327680
NCH = EP // CH
NP = 10112
NC, NS = 2, 16
ZR = NP // NS

_f32 = jnp.float32


def _mesh():
    return plsc.VectorSubcoreMesh(
        core_axis_name="c", subcore_axis_name="s", num_cores=NC, num_subcores=NS
    )


_HW = 64


def _make_spmm():
    ncht = NCH // NS

    @functools.partial(
        pl.kernel,
        out_type=(
            jax.ShapeDtypeStruct((NP, _HW), _f32),
            jax.ShapeDtypeStruct((NP, _HW), _f32),
        ),
        mesh=_mesh(),
        scratch_types=[
            pltpu.VMEM((ncht, CH), jnp.int32),
            pltpu.VMEM((ncht, CH), jnp.int32),
            pltpu.VMEM((CH, _HW), _f32),
            pltpu.VMEM_SHARED((NP, _HW), _f32),
            pltpu.SemaphoreType.DMA,
        ],
        compiler_params=pltpu.CompilerParams(use_tc_tiling_on_sc=False),
    )
    def spmm(tabL, tabR, src2, dst2, zeros, outL, outR,
             srcv, dstv, rows, acc, sem):
        c = lax.axis_index("c")
        s = lax.axis_index("s")
        r0 = s * ZR
        pltpu.sync_copy(zeros.at[pl.ds(r0, ZR)], acc.at[pl.ds(r0, ZR)])
        ch0 = s * ncht
        pltpu.sync_copy(src2.at[pl.ds(ch0, ncht)], srcv)
        pltpu.sync_copy(dst2.at[pl.ds(ch0, ncht)], dstv)
        plsc.subcore_barrier()

        def edge_loop(tab):
            def body(k, carry):
                pltpu.async_copy(tab.at[srcv.at[k]], rows, sem).wait()
                pltpu.sync_copy(rows, acc.at[dstv.at[k]], add=True)
                return carry
            lax.fori_loop(0, ncht, body, 0)

        @pl.when(c == 0)
        def _():
            edge_loop(tabL)

        @pl.when(c == 1)
        def _():
            edge_loop(tabR)

        plsc.subcore_barrier()

        @pl.when(c == 0)
        def _():
            pltpu.sync_copy(acc.at[pl.ds(r0, ZR)], outL.at[pl.ds(r0, ZR)])

        @pl.when(c == 1)
        def _():
            pltpu.sync_copy(acc.at[pl.ds(r0, ZR)], outR.at[pl.ds(r0, ZR)])

    return spmm


_spmm64_kernel = functools.cache(_make_spmm)


def _make_deg():
    ncht = NCH // (NC * NS)
    zr = NP // NS

    @functools.partial(
        pl.kernel,
        out_type=(
            jax.ShapeDtypeStruct((NP, 16), _f32),
            jax.ShapeDtypeStruct((NP, 16), _f32),
        ),
        mesh=_mesh(),
        scratch_types=[
            pltpu.VMEM((ncht, CH), jnp.int32),
            pltpu.VMEM((CH, 16), _f32),
            pltpu.VMEM_SHARED((NP, 16), _f32),
        ],
        compiler_params=pltpu.CompilerParams(use_tc_tiling_on_sc=False),
    )
    def deg(dst2, zeros16, ones16, outA, outB, dstv, ones_v, acc):
        c = lax.axis_index("c")
        s = lax.axis_index("s")
        r0 = s * zr
        pltpu.sync_copy(zeros16.at[pl.ds(r0, zr)], acc.at[pl.ds(r0, zr)])
        pltpu.sync_copy(ones16, ones_v)
        ch0 = (c * NS + s) * ncht
        pltpu.sync_copy(dst2.at[pl.ds(ch0, ncht)], dstv)
        plsc.subcore_barrier()

        def body(k, carry):
            pltpu.sync_copy(ones_v, acc.at[dstv.at[k]], add=True)
            return carry
        lax.fori_loop(0, ncht, body, 0)

        plsc.subcore_barrier()

        @pl.when(c == 0)
        def _():
            pltpu.sync_copy(acc.at[pl.ds(r0, zr)], outA.at[pl.ds(r0, zr)])

        @pl.when(c == 1)
        def _():
            pltpu.sync_copy(acc.at[pl.ds(r0, zr)], outB.at[pl.ds(r0, zr)])

    return deg


_deg_kernel = functools.cache(_make_deg)


_BR = 1000
_GRID = N // _BR


def _row_spec(w):
    return pl.BlockSpec((_BR, w), lambda i: (i, 0))


def _full_spec(r, w):
    return pl.BlockSpec((r, w), lambda i: (0, 0))


def _layer(h_parts, q_parts, dega, degb, W, b, out_width):
    nh, nq = len(h_parts), len(q_parts)

    def body(*refs):
        h_refs = refs[:nh]
        q_refs = refs[nh:nh + nq]
        da_ref, db_ref, w_ref, b_ref = refs[nh + nq:nh + nq + 4]
        outs = refs[nh + nq + 4:]
        deg = da_ref[:, 0:1] + db_ref[:, 0:1]
        dinv = 1.0 / jnp.maximum(deg, 1.0)
        z = b_ref[...]
        for p, h_ref in enumerate(h_refs):
            z = z + jnp.dot(h_ref[...], w_ref[p * D:(p + 1) * D, :],
                            preferred_element_type=_f32)
        base = nh * D
        for p, q_ref in enumerate(q_refs):
            z = z + jnp.dot(q_ref[...] * dinv,
                            w_ref[base + p * _HW:base + (p + 1) * _HW, :],
                            preferred_element_type=_f32)
        z = jnp.maximum(z, 0.0)
        if len(outs) == 2:
            outs[0][...] = z[:, :D]
            outs[1][...] = z[:, D:]
        else:
            outs[0][...] = z

    n_out = out_width // D
    out_shape = tuple(jax.ShapeDtypeStruct((N, D), _f32) for _ in range(n_out))
    return pl.pallas_call(
        body,
        grid=(_GRID,),
        in_specs=(
            [_row_spec(D)] * nh + [_row_spec(_HW)] * nq
            + [_row_spec(16), _row_spec(16),
               _full_spec(nh * D + nq * _HW, out_width),
               _full_spec(1, out_width)]
        ),
        out_specs=tuple(_row_spec(D) for _ in range(n_out)),
        out_shape=out_shape,
    )(*h_parts, *q_parts, dega, degb, W, b)


def _classifier(h, Wc1, bc1, a, Wc2p, bc2p):
    def body(h_ref, w1_ref, b1_ref, a_ref, w2_ref, b2_ref, out_ref):
        z = jnp.dot(h_ref[...], w1_ref[...], preferred_element_type=_f32) \
            + b1_ref[...]
        z = jnp.where(z > 0.0, z, a_ref[...] * z)
        logits = jnp.dot(z, w2_ref[...], preferred_element_type=_f32) \
            + b2_ref[...]
        m = jnp.max(logits, axis=-1, keepdims=True)
        lse = m + jnp.log(jnp.sum(jnp.exp(logits - m), axis=-1, keepdims=True))
        out_ref[...] = logits - lse

    return pl.pallas_call(
        body,
        grid=(_GRID,),
        in_specs=[
            _row_spec(D),
            _full_spec(D, D), _full_spec(1, D), _full_spec(1, D),
            _full_spec(D, D), _full_spec(1, D),
        ],
        out_specs=_row_spec(D),
        out_shape=jax.ShapeDtypeStruct((N, D), _f32),
    )(h, Wc1, bc1, a, Wc2p, bc2p)


def kernel(x, edge_index, W1, b1, W2, b2, W3, b3, W4, b4,
           Wc1, bc1, prelu_a, Wc2, bc2):
    src = edge_index[0]
    dst = edge_index[1]
    pad = EP - E
    src2 = jnp.concatenate([src, jnp.zeros((pad,), jnp.int32)]).reshape(NCH, CH)
    dst2 = jnp.concatenate([dst, jnp.full((pad,), N, jnp.int32)]).reshape(NCH, CH)
    zNP = jnp.zeros((NP, _HW), _f32)
    z16 = jnp.zeros((NP, 16), _f32)
    o16 = jnp.ones((CH, 16), _f32)

    dega, degb = _deg_kernel()(dst2, z16, o16)

    q1a, q1b = _spmm64_kernel()(x[:, :_HW], x[:, _HW:], src2, dst2, zNP)
    h1L, h1R = _layer([x], [q1a, q1b], dega, degb, W1,
                      b1.reshape(1, -1), 256)

    q2a, q2b = _spmm64_kernel()(h1L[:, :_HW], h1L[:, _HW:], src2, dst2, zNP)
    q2c, q2d = _spmm64_kernel()(h1R[:, :_HW], h1R[:, _HW:], src2, dst2, zNP)
    h2L, h2R = _layer([h1L, h1R], [q2a, q2b, q2c, q2d], dega, degb, W2,
                      b2.reshape(1, -1), 256)

    q3a, q3b = _spmm64_kernel()(h2L[:, :_HW], h2L[:, _HW:], src2, dst2, zNP)
    q3c, q3d = _spmm64_kernel()(h2R[:, :_HW], h2R[:, _HW:], src2, dst2, zNP)
    (h3,) = _layer([h2L, h2R], [q3a, q3b, q3c, q3d], dega, degb, W3,
                   b3.reshape(1, -1), 128)

    q4a, q4b = _spmm64_kernel()(h3[:, :_HW], h3[:, _HW:], src2, dst2, zNP)
    (h4,) = _layer([h3], [q4a, q4b], dega, degb, W4,
                   b4.reshape(1, -1), 128)

    Wc2p = jnp.zeros((D, D), _f32).at[:, :2].set(Wc2)
    bc2p = jnp.full((1, D), -1e30, _f32).at[0, :2].set(bc2)
    outp = _classifier(h4, Wc1, bc1.reshape(1, -1),
                       prelu_a.reshape(1, -1), Wc2p, bc2p)
    return outp[:, :2]


# speedup vs baseline: 3.6240x; 1.3030x over previous
import functools

import jax
import jax.numpy as jnp
from jax import lax
from jax.experimental import pallas as pl
from jax.experimental.pallas import tpu as pltpu
from jax.experimental.pallas import tpu_sc as plsc

N = 10000
E = 320000
D = 128
CH = 128
EP = 327680
NCH = EP // CH
NP = 10112
NC, NS = 2, 16
ZR = NP // NS

_f32 = jnp.float32


def _mesh():
    return plsc.VectorSubcoreMesh(
        core_axis_name="c", subcore_axis_name="s", num_cores=NC, num_subcores=NS
    )


_HW = 64


def _make_spmm():
    ncht = NCH // NS

    @functools.partial(
        pl.kernel,
        out_type=(
            jax.ShapeDtypeStruct((NP, _HW), _f32),
            jax.ShapeDtypeStruct((NP, _HW), _f32),
        ),
        mesh=_mesh(),
        scratch_types=[
            pltpu.VMEM((ncht, CH), jnp.int32),
            pltpu.VMEM((ncht, CH), jnp.int32),
            pltpu.VMEM((CH, _HW), _f32),
            pltpu.VMEM((CH, _HW), _f32),
            pltpu.VMEM_SHARED((NP, _HW), _f32),
            pltpu.SemaphoreType.DMA,
            pltpu.SemaphoreType.DMA,
        ],
        compiler_params=pltpu.CompilerParams(use_tc_tiling_on_sc=False),
    )
    def spmm(tabL, tabR, src2, dst2, zeros, outL, outR,
             srcv, dstv, rows0, rows1, acc, sem0, sem1):
        c = lax.axis_index("c")
        s = lax.axis_index("s")
        r0 = s * ZR
        pltpu.sync_copy(zeros.at[pl.ds(r0, ZR)], acc.at[pl.ds(r0, ZR)])
        ch0 = s * ncht
        pltpu.sync_copy(src2.at[pl.ds(ch0, ncht)], srcv)
        pltpu.sync_copy(dst2.at[pl.ds(ch0, ncht)], dstv)
        plsc.subcore_barrier()

        def edge_loop(tab):
            pltpu.async_copy(tab.at[srcv.at[0]], rows0, sem0)

            def body(i, carry):
                k0 = 2 * i
                k1 = 2 * i + 1
                pltpu.async_copy(tab.at[srcv.at[k1]], rows1, sem1)
                pltpu.make_async_copy(tab.at[srcv.at[k0]], rows0, sem0).wait()
                pltpu.sync_copy(rows0, acc.at[dstv.at[k0]], add=True)

                @pl.when(k0 + 2 < ncht)
                def _():
                    pltpu.async_copy(tab.at[srcv.at[k0 + 2]], rows0, sem0)

                pltpu.make_async_copy(tab.at[srcv.at[k1]], rows1, sem1).wait()
                pltpu.sync_copy(rows1, acc.at[dstv.at[k1]], add=True)
                return carry
            lax.fori_loop(0, ncht // 2, body, 0)

        @pl.when(c == 0)
        def _():
            edge_loop(tabL)

        @pl.when(c == 1)
        def _():
            edge_loop(tabR)

        plsc.subcore_barrier()

        @pl.when(c == 0)
        def _():
            pltpu.sync_copy(acc.at[pl.ds(r0, ZR)], outL.at[pl.ds(r0, ZR)])

        @pl.when(c == 1)
        def _():
            pltpu.sync_copy(acc.at[pl.ds(r0, ZR)], outR.at[pl.ds(r0, ZR)])

    return spmm


_spmm64_kernel = functools.cache(_make_spmm)


def _make_deg():
    ncht = NCH // (NC * NS)
    zr = NP // NS

    @functools.partial(
        pl.kernel,
        out_type=(
            jax.ShapeDtypeStruct((NP, 16), _f32),
            jax.ShapeDtypeStruct((NP, 16), _f32),
        ),
        mesh=_mesh(),
        scratch_types=[
            pltpu.VMEM((ncht, CH), jnp.int32),
            pltpu.VMEM((CH, 16), _f32),
            pltpu.VMEM_SHARED((NP, 16), _f32),
        ],
        compiler_params=pltpu.CompilerParams(use_tc_tiling_on_sc=False),
    )
    def deg(dst2, zeros16, ones16, outA, outB, dstv, ones_v, acc):
        c = lax.axis_index("c")
        s = lax.axis_index("s")
        r0 = s * zr
        pltpu.sync_copy(zeros16.at[pl.ds(r0, zr)], acc.at[pl.ds(r0, zr)])
        pltpu.sync_copy(ones16, ones_v)
        ch0 = (c * NS + s) * ncht
        pltpu.sync_copy(dst2.at[pl.ds(ch0, ncht)], dstv)
        plsc.subcore_barrier()

        def body(k, carry):
            pltpu.sync_copy(ones_v, acc.at[dstv.at[k]], add=True)
            return carry
        lax.fori_loop(0, ncht, body, 0)

        plsc.subcore_barrier()

        @pl.when(c == 0)
        def _():
            pltpu.sync_copy(acc.at[pl.ds(r0, zr)], outA.at[pl.ds(r0, zr)])

        @pl.when(c == 1)
        def _():
            pltpu.sync_copy(acc.at[pl.ds(r0, zr)], outB.at[pl.ds(r0, zr)])

    return deg


_deg_kernel = functools.cache(_make_deg)


_BR = 1000
_GRID = N // _BR


def _row_spec(w):
    return pl.BlockSpec((_BR, w), lambda i: (i, 0))


def _full_spec(r, w):
    return pl.BlockSpec((r, w), lambda i: (0, 0))


def _layer(h_parts, q_parts, dega, degb, W, b, out_width):
    nh, nq = len(h_parts), len(q_parts)

    def body(*refs):
        h_refs = refs[:nh]
        q_refs = refs[nh:nh + nq]
        da_ref, db_ref, w_ref, b_ref = refs[nh + nq:nh + nq + 4]
        outs = refs[nh + nq + 4:]
        deg = da_ref[:, 0:1] + db_ref[:, 0:1]
        dinv = 1.0 / jnp.maximum(deg, 1.0)
        z = b_ref[...]
        for p, h_ref in enumerate(h_refs):
            z = z + jnp.dot(h_ref[...], w_ref[p * D:(p + 1) * D, :],
                            preferred_element_type=_f32)
        base = nh * D
        for p, q_ref in enumerate(q_refs):
            z = z + jnp.dot(q_ref[...] * dinv,
                            w_ref[base + p * _HW:base + (p + 1) * _HW, :],
                            preferred_element_type=_f32)
        z = jnp.maximum(z, 0.0)
        if len(outs) == 2:
            outs[0][...] = z[:, :D]
            outs[1][...] = z[:, D:]
        else:
            outs[0][...] = z

    n_out = out_width // D
    out_shape = tuple(jax.ShapeDtypeStruct((N, D), _f32) for _ in range(n_out))
    return pl.pallas_call(
        body,
        grid=(_GRID,),
        in_specs=(
            [_row_spec(D)] * nh + [_row_spec(_HW)] * nq
            + [_row_spec(16), _row_spec(16),
               _full_spec(nh * D + nq * _HW, out_width),
               _full_spec(1, out_width)]
        ),
        out_specs=tuple(_row_spec(D) for _ in range(n_out)),
        out_shape=out_shape,
    )(*h_parts, *q_parts, dega, degb, W, b)


def _classifier(h, Wc1, bc1, a, Wc2p, bc2p):
    def body(h_ref, w1_ref, b1_ref, a_ref, w2_ref, b2_ref, out_ref):
        z = jnp.dot(h_ref[...], w1_ref[...], preferred_element_type=_f32) \
            + b1_ref[...]
        z = jnp.where(z > 0.0, z, a_ref[...] * z)
        logits = jnp.dot(z, w2_ref[...], preferred_element_type=_f32) \
            + b2_ref[...]
        m = jnp.max(logits, axis=-1, keepdims=True)
        lse = m + jnp.log(jnp.sum(jnp.exp(logits - m), axis=-1, keepdims=True))
        out_ref[...] = logits - lse

    return pl.pallas_call(
        body,
        grid=(_GRID,),
        in_specs=[
            _row_spec(D),
            _full_spec(D, D), _full_spec(1, D), _full_spec(1, D),
            _full_spec(D, D), _full_spec(1, D),
        ],
        out_specs=_row_spec(D),
        out_shape=jax.ShapeDtypeStruct((N, D), _f32),
    )(h, Wc1, bc1, a, Wc2p, bc2p)


def kernel(x, edge_index, W1, b1, W2, b2, W3, b3, W4, b4,
           Wc1, bc1, prelu_a, Wc2, bc2):
    src = edge_index[0]
    dst = edge_index[1]
    pad = EP - E
    src2 = jnp.concatenate([src, jnp.zeros((pad,), jnp.int32)]).reshape(NCH, CH)
    dst2 = jnp.concatenate([dst, jnp.full((pad,), N, jnp.int32)]).reshape(NCH, CH)
    zNP = jnp.zeros((NP, _HW), _f32)
    z16 = jnp.zeros((NP, 16), _f32)
    o16 = jnp.ones((CH, 16), _f32)

    dega, degb = _deg_kernel()(dst2, z16, o16)

    q1a, q1b = _spmm64_kernel()(x[:, :_HW], x[:, _HW:], src2, dst2, zNP)
    h1L, h1R = _layer([x], [q1a, q1b], dega, degb, W1,
                      b1.reshape(1, -1), 256)

    q2a, q2b = _spmm64_kernel()(h1L[:, :_HW], h1L[:, _HW:], src2, dst2, zNP)
    q2c, q2d = _spmm64_kernel()(h1R[:, :_HW], h1R[:, _HW:], src2, dst2, zNP)
    h2L, h2R = _layer([h1L, h1R], [q2a, q2b, q2c, q2d], dega, degb, W2,
                      b2.reshape(1, -1), 256)

    q3a, q3b = _spmm64_kernel()(h2L[:, :_HW], h2L[:, _HW:], src2, dst2, zNP)
    q3c, q3d = _spmm64_kernel()(h2R[:, :_HW], h2R[:, _HW:], src2, dst2, zNP)
    (h3,) = _layer([h2L, h2R], [q3a, q3b, q3c, q3d], dega, degb, W3,
                   b3.reshape(1, -1), 128)

    q4a, q4b = _spmm64_kernel()(h3[:, :_HW], h3[:, _HW:], src2, dst2, zNP)
    (h4,) = _layer([h3], [q4a, q4b], dega, degb, W4,
                   b4.reshape(1, -1), 128)

    Wc2p = jnp.zeros((D, D), _f32).at[:, :2].set(Wc2)
    bc2p = jnp.full((1, D), -1e30, _f32).at[0, :2].set(bc2)
    outp = _classifier(h4, Wc1, bc1.reshape(1, -1),
                       prelu_a.reshape(1, -1), Wc2p, bc2p)
    return outp[:, :2]


# speedup vs baseline: 3.7237x; 1.0275x over previous
import functools

import jax
import jax.numpy as jnp
from jax import lax
from jax.experimental import pallas as pl
from jax.experimental.pallas import tpu as pltpu
from jax.experimental.pallas import tpu_sc as plsc

N = 10000
E = 320000
D = 128
CH = 128
EP = 327680
NCH = EP // CH
NP = 10112
NC, NS = 2, 16
ZR = NP // NS

_f32 = jnp.float32


def _mesh():
    return plsc.VectorSubcoreMesh(
        core_axis_name="c", subcore_axis_name="s", num_cores=NC, num_subcores=NS
    )


_HW = 64


def _make_spmm():
    ncht = NCH // NS

    @functools.partial(
        pl.kernel,
        out_type=(
            jax.ShapeDtypeStruct((NP, _HW), _f32),
            jax.ShapeDtypeStruct((NP, _HW), _f32),
        ),
        mesh=_mesh(),
        scratch_types=[
            pltpu.VMEM((ncht, CH), jnp.int32),
            pltpu.VMEM((ncht, CH), jnp.int32),
            pltpu.VMEM((CH, _HW), _f32),
            pltpu.VMEM((CH, _HW), _f32),
            pltpu.VMEM((CH, _HW), _f32),
            pltpu.VMEM((CH, _HW), _f32),
            pltpu.VMEM_SHARED((NP, _HW), _f32),
        ] + [pltpu.SemaphoreType.DMA] * 8,
        compiler_params=pltpu.CompilerParams(use_tc_tiling_on_sc=False),
    )
    def spmm(tabL, tabR, src2, dst2, zeros, outL, outR,
             srcv, dstv, rows0, rows1, rows2, rows3, acc,
             gs0, gs1, gs2, gs3, ss0, ss1, ss2, ss3):
        c = lax.axis_index("c")
        s = lax.axis_index("s")
        r0 = s * ZR
        pltpu.sync_copy(zeros.at[pl.ds(r0, ZR)], acc.at[pl.ds(r0, ZR)])
        ch0 = s * ncht
        pltpu.sync_copy(src2.at[pl.ds(ch0, ncht)], srcv)
        pltpu.sync_copy(dst2.at[pl.ds(ch0, ncht)], dstv)
        plsc.subcore_barrier()

        rows = [rows0, rows1, rows2, rows3]
        gs = [gs0, gs1, gs2, gs3]
        ss = [ss0, ss1, ss2, ss3]

        def edge_loop(tab):
            pltpu.async_copy(tab.at[srcv.at[0]], rows[0], gs[0])
            pltpu.async_copy(tab.at[srcv.at[1]], rows[1], gs[1])

            def body(i, carry):
                for j in range(4):
                    k = 4 * i + j
                    bg = (j + 2) % 4

                    @pl.when(k >= 2)
                    def _(bg=bg, k=k):
                        pltpu.make_async_copy(
                            rows[bg], acc.at[dstv.at[k - 2]], ss[bg]).wait()

                    @pl.when(k + 2 < ncht)
                    def _(bg=bg, k=k):
                        pltpu.async_copy(tab.at[srcv.at[k + 2]], rows[bg],
                                         gs[bg])

                    pltpu.make_async_copy(tab.at[srcv.at[k]], rows[j],
                                          gs[j]).wait()
                    pltpu.async_copy(rows[j], acc.at[dstv.at[k]], ss[j],
                                     add=True)
                return carry
            lax.fori_loop(0, ncht // 4, body, 0)
            pltpu.make_async_copy(rows[2], acc.at[dstv.at[ncht - 2]],
                                  ss[2]).wait()
            pltpu.make_async_copy(rows[3], acc.at[dstv.at[ncht - 1]],
                                  ss[3]).wait()

        @pl.when(c == 0)
        def _():
            edge_loop(tabL)

        @pl.when(c == 1)
        def _():
            edge_loop(tabR)

        plsc.subcore_barrier()

        @pl.when(c == 0)
        def _():
            pltpu.sync_copy(acc.at[pl.ds(r0, ZR)], outL.at[pl.ds(r0, ZR)])

        @pl.when(c == 1)
        def _():
            pltpu.sync_copy(acc.at[pl.ds(r0, ZR)], outR.at[pl.ds(r0, ZR)])

    return spmm


_spmm64_kernel = functools.cache(_make_spmm)


def _make_deg():
    ncht = NCH // (NC * NS)
    zr = NP // NS

    @functools.partial(
        pl.kernel,
        out_type=(
            jax.ShapeDtypeStruct((NP, 16), _f32),
            jax.ShapeDtypeStruct((NP, 16), _f32),
        ),
        mesh=_mesh(),
        scratch_types=[
            pltpu.VMEM((ncht, CH), jnp.int32),
            pltpu.VMEM((CH, 16), _f32),
            pltpu.VMEM_SHARED((NP, 16), _f32),
        ],
        compiler_params=pltpu.CompilerParams(use_tc_tiling_on_sc=False),
    )
    def deg(dst2, zeros16, ones16, outA, outB, dstv, ones_v, acc):
        c = lax.axis_index("c")
        s = lax.axis_index("s")
        r0 = s * zr
        pltpu.sync_copy(zeros16.at[pl.ds(r0, zr)], acc.at[pl.ds(r0, zr)])
        pltpu.sync_copy(ones16, ones_v)
        ch0 = (c * NS + s) * ncht
        pltpu.sync_copy(dst2.at[pl.ds(ch0, ncht)], dstv)
        plsc.subcore_barrier()

        def body(k, carry):
            pltpu.sync_copy(ones_v, acc.at[dstv.at[k]], add=True)
            return carry
        lax.fori_loop(0, ncht, body, 0)

        plsc.subcore_barrier()

        @pl.when(c == 0)
        def _():
            pltpu.sync_copy(acc.at[pl.ds(r0, zr)], outA.at[pl.ds(r0, zr)])

        @pl.when(c == 1)
        def _():
            pltpu.sync_copy(acc.at[pl.ds(r0, zr)], outB.at[pl.ds(r0, zr)])

    return deg


_deg_kernel = functools.cache(_make_deg)


_BR = 1000
_GRID = N // _BR


def _row_spec(w):
    return pl.BlockSpec((_BR, w), lambda i: (i, 0))


def _full_spec(r, w):
    return pl.BlockSpec((r, w), lambda i: (0, 0))


def _layer(h_parts, q_parts, dega, degb, W, b, out_width):
    nh, nq = len(h_parts), len(q_parts)

    def body(*refs):
        h_refs = refs[:nh]
        q_refs = refs[nh:nh + nq]
        da_ref, db_ref, w_ref, b_ref = refs[nh + nq:nh + nq + 4]
        outs = refs[nh + nq + 4:]
        deg = da_ref[:, 0:1] + db_ref[:, 0:1]
        dinv = 1.0 / jnp.maximum(deg, 1.0)
        z = b_ref[...]
        for p, h_ref in enumerate(h_refs):
            z = z + jnp.dot(h_ref[...], w_ref[p * D:(p + 1) * D, :],
                            preferred_element_type=_f32)
        base = nh * D
        for p, q_ref in enumerate(q_refs):
            z = z + jnp.dot(q_ref[...] * dinv,
                            w_ref[base + p * _HW:base + (p + 1) * _HW, :],
                            preferred_element_type=_f32)
        z = jnp.maximum(z, 0.0)
        if len(outs) == 2:
            outs[0][...] = z[:, :D]
            outs[1][...] = z[:, D:]
        else:
            outs[0][...] = z

    n_out = out_width // D
    out_shape = tuple(jax.ShapeDtypeStruct((N, D), _f32) for _ in range(n_out))
    return pl.pallas_call(
        body,
        grid=(_GRID,),
        in_specs=(
            [_row_spec(D)] * nh + [_row_spec(_HW)] * nq
            + [_row_spec(16), _row_spec(16),
               _full_spec(nh * D + nq * _HW, out_width),
               _full_spec(1, out_width)]
        ),
        out_specs=tuple(_row_spec(D) for _ in range(n_out)),
        out_shape=out_shape,
    )(*h_parts, *q_parts, dega, degb, W, b)


def _classifier(h, Wc1, bc1, a, Wc2p, bc2p):
    def body(h_ref, w1_ref, b1_ref, a_ref, w2_ref, b2_ref, out_ref):
        z = jnp.dot(h_ref[...], w1_ref[...], preferred_element_type=_f32) \
            + b1_ref[...]
        z = jnp.where(z > 0.0, z, a_ref[...] * z)
        logits = jnp.dot(z, w2_ref[...], preferred_element_type=_f32) \
            + b2_ref[...]
        m = jnp.max(logits, axis=-1, keepdims=True)
        lse = m + jnp.log(jnp.sum(jnp.exp(logits - m), axis=-1, keepdims=True))
        out_ref[...] = logits - lse

    return pl.pallas_call(
        body,
        grid=(_GRID,),
        in_specs=[
            _row_spec(D),
            _full_spec(D, D), _full_spec(1, D), _full_spec(1, D),
            _full_spec(D, D), _full_spec(1, D),
        ],
        out_specs=_row_spec(D),
        out_shape=jax.ShapeDtypeStruct((N, D), _f32),
    )(h, Wc1, bc1, a, Wc2p, bc2p)


def kernel(x, edge_index, W1, b1, W2, b2, W3, b3, W4, b4,
           Wc1, bc1, prelu_a, Wc2, bc2):
    src = edge_index[0]
    dst = edge_index[1]
    pad = EP - E
    src2 = jnp.concatenate([src, jnp.zeros((pad,), jnp.int32)]).reshape(NCH, CH)
    dst2 = jnp.concatenate([dst, jnp.full((pad,), N, jnp.int32)]).reshape(NCH, CH)
    zNP = jnp.zeros((NP, _HW), _f32)
    z16 = jnp.zeros((NP, 16), _f32)
    o16 = jnp.ones((CH, 16), _f32)

    dega, degb = _deg_kernel()(dst2, z16, o16)

    q1a, q1b = _spmm64_kernel()(x[:, :_HW], x[:, _HW:], src2, dst2, zNP)
    h1L, h1R = _layer([x], [q1a, q1b], dega, degb, W1,
                      b1.reshape(1, -1), 256)

    q2a, q2b = _spmm64_kernel()(h1L[:, :_HW], h1L[:, _HW:], src2, dst2, zNP)
    q2c, q2d = _spmm64_kernel()(h1R[:, :_HW], h1R[:, _HW:], src2, dst2, zNP)
    h2L, h2R = _layer([h1L, h1R], [q2a, q2b, q2c, q2d], dega, degb, W2,
                      b2.reshape(1, -1), 256)

    q3a, q3b = _spmm64_kernel()(h2L[:, :_HW], h2L[:, _HW:], src2, dst2, zNP)
    q3c, q3d = _spmm64_kernel()(h2R[:, :_HW], h2R[:, _HW:], src2, dst2, zNP)
    (h3,) = _layer([h2L, h2R], [q3a, q3b, q3c, q3d], dega, degb, W3,
                   b3.reshape(1, -1), 128)

    q4a, q4b = _spmm64_kernel()(h3[:, :_HW], h3[:, _HW:], src2, dst2, zNP)
    (h4,) = _layer([h3], [q4a, q4b], dega, degb, W4,
                   b4.reshape(1, -1), 128)

    Wc2p = jnp.zeros((D, D), _f32).at[:, :2].set(Wc2)
    bc2p = jnp.full((1, D), -1e30, _f32).at[0, :2].set(bc2)
    outp = _classifier(h4, Wc1, bc1.reshape(1, -1),
                       prelu_a.reshape(1, -1), Wc2p, bc2p)
    return outp[:, :2]


# speedup vs baseline: 4.0995x; 1.1009x over previous
import functools

import jax
import jax.numpy as jnp
from jax import lax
from jax.experimental import pallas as pl
from jax.experimental.pallas import tpu as pltpu
from jax.experimental.pallas import tpu_sc as plsc

N = 10000
E = 320000
D = 128
CH = 128
EP = 327680
NCH = EP // CH
NP = 10112
NC, NS = 2, 16
ZR = NP // NS

_f32 = jnp.float32


def _mesh():
    return plsc.VectorSubcoreMesh(
        core_axis_name="c", subcore_axis_name="s", num_cores=NC, num_subcores=NS
    )


_HW = 64


def _make_spmm():
    ncht = NCH // NS

    @functools.partial(
        pl.kernel,
        out_type=(
            jax.ShapeDtypeStruct((NP, _HW), _f32),
            jax.ShapeDtypeStruct((NP, _HW), _f32),
        ),
        mesh=_mesh(),
        scratch_types=[
            pltpu.VMEM((ncht, CH), jnp.int32),
            pltpu.VMEM((ncht, CH), jnp.int32),
            pltpu.VMEM((CH, _HW), _f32),
            pltpu.VMEM((CH, _HW), _f32),
            pltpu.VMEM((CH, _HW), _f32),
            pltpu.VMEM((CH, _HW), _f32),
            pltpu.VMEM_SHARED((NP, _HW), _f32),
        ] + [pltpu.SemaphoreType.DMA] * 8,
        compiler_params=pltpu.CompilerParams(use_tc_tiling_on_sc=False),
    )
    def spmm(tabL, tabR, src2, dst2, zeros, outL, outR,
             srcv, dstv, rows0, rows1, rows2, rows3, acc,
             gs0, gs1, gs2, gs3, ss0, ss1, ss2, ss3):
        c = lax.axis_index("c")
        s = lax.axis_index("s")
        r0 = s * ZR
        pltpu.sync_copy(zeros.at[pl.ds(r0, ZR)], acc.at[pl.ds(r0, ZR)])
        ch0 = s * ncht
        pltpu.sync_copy(src2.at[pl.ds(ch0, ncht)], srcv)
        pltpu.sync_copy(dst2.at[pl.ds(ch0, ncht)], dstv)
        plsc.subcore_barrier()

        rows = [rows0, rows1, rows2, rows3]
        gs = [gs0, gs1, gs2, gs3]
        ss = [ss0, ss1, ss2, ss3]

        def edge_loop(tab):
            pltpu.async_copy(tab.at[srcv.at[0]], rows[0], gs[0])
            pltpu.async_copy(tab.at[srcv.at[1]], rows[1], gs[1])

            def body(i, carry):
                for j in range(4):
                    k = 4 * i + j
                    bg = (j + 2) % 4

                    @pl.when(k >= 2)
                    def _(bg=bg, k=k):
                        pltpu.make_async_copy(
                            rows[bg], acc.at[dstv.at[k - 2]], ss[bg]).wait()

                    @pl.when(k + 2 < ncht)
                    def _(bg=bg, k=k):
                        pltpu.async_copy(tab.at[srcv.at[k + 2]], rows[bg],
                                         gs[bg])

                    pltpu.make_async_copy(tab.at[srcv.at[k]], rows[j],
                                          gs[j]).wait()
                    pltpu.async_copy(rows[j], acc.at[dstv.at[k]], ss[j],
                                     add=True)
                return carry
            lax.fori_loop(0, ncht // 4, body, 0)
            pltpu.make_async_copy(rows[2], acc.at[dstv.at[ncht - 2]],
                                  ss[2]).wait()
            pltpu.make_async_copy(rows[3], acc.at[dstv.at[ncht - 1]],
                                  ss[3]).wait()

        @pl.when(c == 0)
        def _():
            edge_loop(tabL)

        @pl.when(c == 1)
        def _():
            edge_loop(tabR)

        plsc.subcore_barrier()

        @pl.when(c == 0)
        def _():
            pltpu.sync_copy(acc.at[pl.ds(r0, ZR)], outL.at[pl.ds(r0, ZR)])

        @pl.when(c == 1)
        def _():
            pltpu.sync_copy(acc.at[pl.ds(r0, ZR)], outR.at[pl.ds(r0, ZR)])

    return spmm


_spmm64_kernel = functools.cache(_make_spmm)


def _make_deg():
    ncht = NCH // (NC * NS)
    zr = NP // NS

    @functools.partial(
        pl.kernel,
        out_type=(
            jax.ShapeDtypeStruct((NP, 16), _f32),
            jax.ShapeDtypeStruct((NP, 16), _f32),
        ),
        mesh=_mesh(),
        scratch_types=[
            pltpu.VMEM((ncht, CH), jnp.int32),
            pltpu.VMEM((CH, 16), _f32),
            pltpu.VMEM_SHARED((NP, 16), _f32),
        ],
        compiler_params=pltpu.CompilerParams(use_tc_tiling_on_sc=False),
    )
    def deg(dst2, zeros16, ones16, outA, outB, dstv, ones_v, acc):
        c = lax.axis_index("c")
        s = lax.axis_index("s")
        r0 = s * zr
        pltpu.sync_copy(zeros16.at[pl.ds(r0, zr)], acc.at[pl.ds(r0, zr)])
        pltpu.sync_copy(ones16, ones_v)
        ch0 = (c * NS + s) * ncht
        pltpu.sync_copy(dst2.at[pl.ds(ch0, ncht)], dstv)
        plsc.subcore_barrier()

        def body(k, carry):
            pltpu.sync_copy(ones_v, acc.at[dstv.at[k]], add=True)
            return carry
        lax.fori_loop(0, ncht, body, 0)

        plsc.subcore_barrier()

        @pl.when(c == 0)
        def _():
            pltpu.sync_copy(acc.at[pl.ds(r0, zr)], outA.at[pl.ds(r0, zr)])

        @pl.when(c == 1)
        def _():
            pltpu.sync_copy(acc.at[pl.ds(r0, zr)], outB.at[pl.ds(r0, zr)])

    return deg


_deg_kernel = functools.cache(_make_deg)


_BR = 1000
_GRID = N // _BR


def _row_spec(w):
    return pl.BlockSpec((_BR, w), lambda i: (i, 0))


def _full_spec(r, w):
    return pl.BlockSpec((r, w), lambda i: (0, 0))


def _layer(h_parts, q_parts, dega, degb, W, b, out_width):
    nh, nq = len(h_parts), len(q_parts)

    def body(*refs):
        h_refs = refs[:nh]
        q_refs = refs[nh:nh + nq]
        da_ref, db_ref, w_ref, b_ref = refs[nh + nq:nh + nq + 4]
        outs = refs[nh + nq + 4:]
        deg = da_ref[:, 0:1] + db_ref[:, 0:1]
        dinv = 1.0 / jnp.maximum(deg, 1.0)
        z = b_ref[...]
        for p, h_ref in enumerate(h_refs):
            z = z + jnp.dot(h_ref[...], w_ref[p * D:(p + 1) * D, :],
                            preferred_element_type=_f32)
        base = nh * D
        for p, q_ref in enumerate(q_refs):
            z = z + jnp.dot(q_ref[...] * dinv,
                            w_ref[base + p * _HW:base + (p + 1) * _HW, :],
                            preferred_element_type=_f32)
        z = jnp.maximum(z, 0.0)
        if len(outs) == 2:
            outs[0][...] = z[:, :D]
            outs[1][...] = z[:, D:]
        else:
            outs[0][...] = z

    n_out = out_width // D
    out_shape = tuple(jax.ShapeDtypeStruct((N, D), _f32) for _ in range(n_out))
    return pl.pallas_call(
        body,
        grid=(_GRID,),
        in_specs=(
            [_row_spec(D)] * nh + [_row_spec(_HW)] * nq
            + [_row_spec(16), _row_spec(16),
               _full_spec(nh * D + nq * _HW, out_width),
               _full_spec(1, out_width)]
        ),
        out_specs=tuple(_row_spec(D) for _ in range(n_out)),
        out_shape=out_shape,
    )(*h_parts, *q_parts, dega, degb, W, b)


def _proj3(hL, hR, W, b):
    def body(hL_ref, hR_ref, w_ref, b_ref, t_ref, pa_ref, pb_ref):
        hl = hL_ref[...]
        hr = hR_ref[...]
        t_ref[...] = (jnp.dot(hl, w_ref[0:D, :], preferred_element_type=_f32)
                      + jnp.dot(hr, w_ref[D:2 * D, :],
                                preferred_element_type=_f32)
                      + b_ref[...])
        p = (jnp.dot(hl, w_ref[2 * D:3 * D, :], preferred_element_type=_f32)
             + jnp.dot(hr, w_ref[3 * D:4 * D, :], preferred_element_type=_f32))
        pa_ref[...] = p[:, :_HW]
        pb_ref[...] = p[:, _HW:]

    return pl.pallas_call(
        body,
        grid=(_GRID,),
        in_specs=[_row_spec(D), _row_spec(D),
                  _full_spec(4 * D, D), _full_spec(1, D)],
        out_specs=(_row_spec(D), _row_spec(_HW), _row_spec(_HW)),
        out_shape=(jax.ShapeDtypeStruct((N, D), _f32),
                   jax.ShapeDtypeStruct((N, _HW), _f32),
                   jax.ShapeDtypeStruct((N, _HW), _f32)),
    )(hL, hR, W, b)


def _fin3(t, qa, qb, dega, degb):
    def body(t_ref, qa_ref, qb_ref, da_ref, db_ref, out_ref):
        deg = da_ref[:, 0:1] + db_ref[:, 0:1]
        dinv = 1.0 / jnp.maximum(deg, 1.0)
        q = jnp.concatenate([qa_ref[...], qb_ref[...]], axis=1)
        out_ref[...] = jnp.maximum(t_ref[...] + q * dinv, 0.0)

    return pl.pallas_call(
        body,
        grid=(_GRID,),
        in_specs=[_row_spec(D), _row_spec(_HW), _row_spec(_HW),
                  _row_spec(16), _row_spec(16)],
        out_specs=_row_spec(D),
        out_shape=jax.ShapeDtypeStruct((N, D), _f32),
    )(t, qa, qb, dega, degb)


def _classifier(h, Wc1, bc1, a, Wc2p, bc2p):
    def body(h_ref, w1_ref, b1_ref, a_ref, w2_ref, b2_ref, out_ref):
        z = jnp.dot(h_ref[...], w1_ref[...], preferred_element_type=_f32) \
            + b1_ref[...]
        z = jnp.where(z > 0.0, z, a_ref[...] * z)
        logits = jnp.dot(z, w2_ref[...], preferred_element_type=_f32) \
            + b2_ref[...]
        m = jnp.max(logits, axis=-1, keepdims=True)
        lse = m + jnp.log(jnp.sum(jnp.exp(logits - m), axis=-1, keepdims=True))
        out_ref[...] = logits - lse

    return pl.pallas_call(
        body,
        grid=(_GRID,),
        in_specs=[
            _row_spec(D),
            _full_spec(D, D), _full_spec(1, D), _full_spec(1, D),
            _full_spec(D, D), _full_spec(1, D),
        ],
        out_specs=_row_spec(D),
        out_shape=jax.ShapeDtypeStruct((N, D), _f32),
    )(h, Wc1, bc1, a, Wc2p, bc2p)


def kernel(x, edge_index, W1, b1, W2, b2, W3, b3, W4, b4,
           Wc1, bc1, prelu_a, Wc2, bc2):
    src = edge_index[0]
    dst = edge_index[1]
    pad = EP - E
    src2 = jnp.concatenate([src, jnp.zeros((pad,), jnp.int32)]).reshape(NCH, CH)
    dst2 = jnp.concatenate([dst, jnp.full((pad,), N, jnp.int32)]).reshape(NCH, CH)
    zNP = jnp.zeros((NP, _HW), _f32)
    z16 = jnp.zeros((NP, 16), _f32)
    o16 = jnp.ones((CH, 16), _f32)

    dega, degb = _deg_kernel()(dst2, z16, o16)

    q1a, q1b = _spmm64_kernel()(x[:, :_HW], x[:, _HW:], src2, dst2, zNP)
    h1L, h1R = _layer([x], [q1a, q1b], dega, degb, W1,
                      b1.reshape(1, -1), 256)

    q2a, q2b = _spmm64_kernel()(h1L[:, :_HW], h1L[:, _HW:], src2, dst2, zNP)
    q2c, q2d = _spmm64_kernel()(h1R[:, :_HW], h1R[:, _HW:], src2, dst2, zNP)
    h2L, h2R = _layer([h1L, h1R], [q2a, q2b, q2c, q2d], dega, degb, W2,
                      b2.reshape(1, -1), 256)

    t3, p3a, p3b = _proj3(h2L, h2R, W3, b3.reshape(1, -1))
    q3a, q3b = _spmm64_kernel()(p3a, p3b, src2, dst2, zNP)
    h3 = _fin3(t3, q3a, q3b, dega, degb)

    q4a, q4b = _spmm64_kernel()(h3[:, :_HW], h3[:, _HW:], src2, dst2, zNP)
    (h4,) = _layer([h3], [q4a, q4b], dega, degb, W4,
                   b4.reshape(1, -1), 128)

    Wc2p = jnp.zeros((D, D), _f32).at[:, :2].set(Wc2)
    bc2p = jnp.full((1, D), -1e30, _f32).at[0, :2].set(bc2)
    outp = _classifier(h4, Wc1, bc1.reshape(1, -1),
                       prelu_a.reshape(1, -1), Wc2p, bc2p)
    return outp[:, :2]


# speedup vs baseline: 8.5377x; 2.0826x over previous
import functools

import jax
import jax.numpy as jnp
from jax import lax
from jax.experimental import pallas as pl
from jax.experimental.pallas import tpu as pltpu
from jax.experimental.pallas import tpu_sc as plsc

N = 10000
E = 320000
D = 128
CH = 128
EP = 327680
NCH = EP // CH
NP = 10112
NC, NS = 2, 16
ZR = NP // NS

_f32 = jnp.float32


def _mesh():
    return plsc.VectorSubcoreMesh(
        core_axis_name="c", subcore_axis_name="s", num_cores=NC, num_subcores=NS
    )


_HW = 64


def _make_spmm():
    ncht = NCH // NS

    @functools.partial(
        pl.kernel,
        out_type=(
            jax.ShapeDtypeStruct((NP, _HW), _f32),
            jax.ShapeDtypeStruct((NP, _HW), _f32),
        ),
        mesh=_mesh(),
        scratch_types=[
            pltpu.VMEM((2, CH), jnp.int32),
            pltpu.VMEM((2, CH), jnp.int32),
            pltpu.VMEM((2, CH), jnp.int32),
            pltpu.VMEM((2, CH), jnp.int32),
            pltpu.VMEM((CH, _HW), _f32),
            pltpu.VMEM((CH, _HW), _f32),
            pltpu.VMEM((CH, _HW), _f32),
            pltpu.VMEM((CH, _HW), _f32),
            pltpu.VMEM_SHARED((NP, _HW), _f32),
            pltpu.VMEM_SHARED((NP, _HW), _f32),
        ] + [pltpu.SemaphoreType.DMA] * 12,
        compiler_params=pltpu.CompilerParams(use_tc_tiling_on_sc=False),
    )
    def spmm(tabL, tabR, edges, zeros, outL, outR,
             eb0, eb1, eb2, eb3, rows0, rows1, rows2, rows3, tab_s, acc,
             es0, es1, es2, es3, gs0, gs1, gs2, gs3, ss0, ss1, ss2, ss3):
        c = lax.axis_index("c")
        s = lax.axis_index("s")
        r0 = s * ZR
        pltpu.sync_copy(zeros.at[pl.ds(r0, ZR)], acc.at[pl.ds(r0, ZR)])

        @pl.when(c == 0)
        def _():
            pltpu.sync_copy(tabL.at[pl.ds(r0, ZR)], tab_s.at[pl.ds(r0, ZR)])

        @pl.when(c == 1)
        def _():
            pltpu.sync_copy(tabR.at[pl.ds(r0, ZR)], tab_s.at[pl.ds(r0, ZR)])

        ch0 = s * ncht
        plsc.subcore_barrier()

        ebuf = [eb0, eb1, eb2, eb3]
        rows = [rows0, rows1, rows2, rows3]
        es = [es0, es1, es2, es3]
        gs = [gs0, gs1, gs2, gs3]
        ss = [ss0, ss1, ss2, ss3]

        pltpu.async_copy(edges.at[ch0], ebuf[0], es[0])
        pltpu.async_copy(edges.at[ch0 + 1], ebuf[1], es[1])
        pltpu.make_async_copy(edges.at[ch0], ebuf[0], es[0]).wait()
        pltpu.async_copy(tab_s.at[ebuf[0].at[0]], rows[0], gs[0])

        def chunk(k, j):
            jn = (j + 2) % 4
            j1 = (j + 1) % 4

            @pl.when(k >= 2)
            def _():
                pltpu.make_async_copy(rows[jn], acc.at[ebuf[jn].at[1]],
                                      ss[jn]).wait()

            @pl.when(k + 2 < ncht)
            def _():
                pltpu.async_copy(edges.at[ch0 + k + 2], ebuf[jn], es[jn])

            @pl.when(k + 1 < ncht)
            def _():
                pltpu.make_async_copy(edges.at[ch0 + k + 1], ebuf[j1],
                                      es[j1]).wait()
                pltpu.async_copy(tab_s.at[ebuf[j1].at[0]], rows[j1], gs[j1])

            pltpu.make_async_copy(tab_s.at[ebuf[j].at[0]], rows[j],
                                  gs[j]).wait()
            pltpu.async_copy(rows[j], acc.at[ebuf[j].at[1]], ss[j], add=True)

        def body4(i, carry):
            for j in range(4):
                chunk(4 * i + j, j)
            return carry
        lax.fori_loop(0, ncht // 4, body4, 0)
        pltpu.make_async_copy(rows[2], acc.at[ebuf[2].at[1]], ss[2]).wait()
        pltpu.make_async_copy(rows[3], acc.at[ebuf[3].at[1]], ss[3]).wait()

        plsc.subcore_barrier()

        @pl.when(c == 0)
        def _():
            pltpu.sync_copy(acc.at[pl.ds(r0, ZR)], outL.at[pl.ds(r0, ZR)])

        @pl.when(c == 1)
        def _():
            pltpu.sync_copy(acc.at[pl.ds(r0, ZR)], outR.at[pl.ds(r0, ZR)])

    return spmm


_spmm64_kernel = functools.cache(_make_spmm)


def _make_deg():
    ncht = NCH // (NC * NS)
    zr = NP // NS

    @functools.partial(
        pl.kernel,
        out_type=(
            jax.ShapeDtypeStruct((NP, 16), _f32),
            jax.ShapeDtypeStruct((NP, 16), _f32),
        ),
        mesh=_mesh(),
        scratch_types=[
            pltpu.VMEM((ncht, CH), jnp.int32),
            pltpu.VMEM((CH, 16), _f32),
            pltpu.VMEM_SHARED((NP, 16), _f32),
        ],
        compiler_params=pltpu.CompilerParams(use_tc_tiling_on_sc=False),
    )
    def deg(dst2, zeros16, ones16, outA, outB, dstv, ones_v, acc):
        c = lax.axis_index("c")
        s = lax.axis_index("s")
        r0 = s * zr
        pltpu.sync_copy(zeros16.at[pl.ds(r0, zr)], acc.at[pl.ds(r0, zr)])
        pltpu.sync_copy(ones16, ones_v)
        ch0 = (c * NS + s) * ncht
        pltpu.sync_copy(dst2.at[pl.ds(ch0, ncht)], dstv)
        plsc.subcore_barrier()

        def body(k, carry):
            pltpu.sync_copy(ones_v, acc.at[dstv.at[k]], add=True)
            return carry
        lax.fori_loop(0, ncht, body, 0)

        plsc.subcore_barrier()

        @pl.when(c == 0)
        def _():
            pltpu.sync_copy(acc.at[pl.ds(r0, zr)], outA.at[pl.ds(r0, zr)])

        @pl.when(c == 1)
        def _():
            pltpu.sync_copy(acc.at[pl.ds(r0, zr)], outB.at[pl.ds(r0, zr)])

    return deg


_deg_kernel = functools.cache(_make_deg)


_BR = 1000
_GRID = N // _BR


def _row_spec(w):
    return pl.BlockSpec((_BR, w), lambda i: (i, 0))


def _full_spec(r, w):
    return pl.BlockSpec((r, w), lambda i: (0, 0))


def _layer(h_parts, q_parts, dega, degb, W, b, out_width):
    nh, nq = len(h_parts), len(q_parts)

    def body(*refs):
        h_refs = refs[:nh]
        q_refs = refs[nh:nh + nq]
        da_ref, db_ref, w_ref, b_ref = refs[nh + nq:nh + nq + 4]
        outs = refs[nh + nq + 4:]
        deg = da_ref[:, 0:1] + db_ref[:, 0:1]
        dinv = 1.0 / jnp.maximum(deg, 1.0)
        z = b_ref[...]
        for p, h_ref in enumerate(h_refs):
            z = z + jnp.dot(h_ref[...], w_ref[p * D:(p + 1) * D, :],
                            preferred_element_type=_f32)
        base = nh * D
        for p, q_ref in enumerate(q_refs):
            z = z + jnp.dot(q_ref[...] * dinv,
                            w_ref[base + p * _HW:base + (p + 1) * _HW, :],
                            preferred_element_type=_f32)
        z = jnp.maximum(z, 0.0)
        if len(outs) == 2:
            outs[0][...] = z[:, :D]
            outs[1][...] = z[:, D:]
        else:
            outs[0][...] = z

    n_out = out_width // D
    out_shape = tuple(jax.ShapeDtypeStruct((NP, D), _f32)
                      for _ in range(n_out))
    return pl.pallas_call(
        body,
        grid=(_GRID,),
        in_specs=(
            [_row_spec(D)] * nh + [_row_spec(_HW)] * nq
            + [_row_spec(16), _row_spec(16),
               _full_spec(nh * D + nq * _HW, out_width),
               _full_spec(1, out_width)]
        ),
        out_specs=tuple(_row_spec(D) for _ in range(n_out)),
        out_shape=out_shape,
    )(*h_parts, *q_parts, dega, degb, W, b)


def _proj3(hL, hR, W, b):
    def body(hL_ref, hR_ref, w_ref, b_ref, t_ref, pa_ref, pb_ref):
        hl = hL_ref[...]
        hr = hR_ref[...]
        t_ref[...] = (jnp.dot(hl, w_ref[0:D, :], preferred_element_type=_f32)
                      + jnp.dot(hr, w_ref[D:2 * D, :],
                                preferred_element_type=_f32)
                      + b_ref[...])
        p = (jnp.dot(hl, w_ref[2 * D:3 * D, :], preferred_element_type=_f32)
             + jnp.dot(hr, w_ref[3 * D:4 * D, :], preferred_element_type=_f32))
        pa_ref[...] = p[:, :_HW]
        pb_ref[...] = p[:, _HW:]

    return pl.pallas_call(
        body,
        grid=(_GRID,),
        in_specs=[_row_spec(D), _row_spec(D),
                  _full_spec(4 * D, D), _full_spec(1, D)],
        out_specs=(_row_spec(D), _row_spec(_HW), _row_spec(_HW)),
        out_shape=(jax.ShapeDtypeStruct((N, D), _f32),
                   jax.ShapeDtypeStruct((NP, _HW), _f32),
                   jax.ShapeDtypeStruct((NP, _HW), _f32)),
    )(hL, hR, W, b)


def _fin3(t, qa, qb, dega, degb):
    def body(t_ref, qa_ref, qb_ref, da_ref, db_ref, out_ref):
        deg = da_ref[:, 0:1] + db_ref[:, 0:1]
        dinv = 1.0 / jnp.maximum(deg, 1.0)
        q = jnp.concatenate([qa_ref[...], qb_ref[...]], axis=1)
        out_ref[...] = jnp.maximum(t_ref[...] + q * dinv, 0.0)

    return pl.pallas_call(
        body,
        grid=(_GRID,),
        in_specs=[_row_spec(D), _row_spec(_HW), _row_spec(_HW),
                  _row_spec(16), _row_spec(16)],
        out_specs=_row_spec(D),
        out_shape=jax.ShapeDtypeStruct((NP, D), _f32),
    )(t, qa, qb, dega, degb)


def _classifier(h, Wc1, bc1, a, Wc2p, bc2p):
    def body(h_ref, w1_ref, b1_ref, a_ref, w2_ref, b2_ref, out_ref):
        z = jnp.dot(h_ref[...], w1_ref[...], preferred_element_type=_f32) \
            + b1_ref[...]
        z = jnp.where(z > 0.0, z, a_ref[...] * z)
        logits = jnp.dot(z, w2_ref[...], preferred_element_type=_f32) \
            + b2_ref[...]
        m = jnp.max(logits, axis=-1, keepdims=True)
        lse = m + jnp.log(jnp.sum(jnp.exp(logits - m), axis=-1, keepdims=True))
        out_ref[...] = logits - lse

    return pl.pallas_call(
        body,
        grid=(_GRID,),
        in_specs=[
            _row_spec(D),
            _full_spec(D, D), _full_spec(1, D), _full_spec(1, D),
            _full_spec(D, D), _full_spec(1, D),
        ],
        out_specs=_row_spec(D),
        out_shape=jax.ShapeDtypeStruct((N, D), _f32),
    )(h, Wc1, bc1, a, Wc2p, bc2p)


def kernel(x, edge_index, W1, b1, W2, b2, W3, b3, W4, b4,
           Wc1, bc1, prelu_a, Wc2, bc2):
    src = edge_index[0]
    dst = edge_index[1]
    pad = EP - E
    src2 = jnp.concatenate([src, jnp.zeros((pad,), jnp.int32)]).reshape(NCH, CH)
    dst2 = jnp.concatenate([dst, jnp.full((pad,), N, jnp.int32)]).reshape(NCH, CH)
    edges = jnp.stack([src2, dst2], axis=1)
    zNP = jnp.zeros((NP, _HW), _f32)
    z16 = jnp.zeros((NP, 16), _f32)
    o16 = jnp.ones((CH, 16), _f32)
    xp = jnp.concatenate([x, jnp.zeros((NP - N, D), _f32)])

    dega, degb = _deg_kernel()(dst2, z16, o16)

    q1a, q1b = _spmm64_kernel()(xp[:, :_HW], xp[:, _HW:], edges, zNP)
    h1L, h1R = _layer([xp], [q1a, q1b], dega, degb, W1,
                      b1.reshape(1, -1), 256)

    q2a, q2b = _spmm64_kernel()(h1L[:, :_HW], h1L[:, _HW:], edges, zNP)
    q2c, q2d = _spmm64_kernel()(h1R[:, :_HW], h1R[:, _HW:], edges, zNP)
    h2L, h2R = _layer([h1L, h1R], [q2a, q2b, q2c, q2d], dega, degb, W2,
                      b2.reshape(1, -1), 256)

    t3, p3a, p3b = _proj3(h2L, h2R, W3, b3.reshape(1, -1))
    q3a, q3b = _spmm64_kernel()(p3a, p3b, edges, zNP)
    h3 = _fin3(t3, q3a, q3b, dega, degb)

    q4a, q4b = _spmm64_kernel()(h3[:, :_HW], h3[:, _HW:], edges, zNP)
    (h4,) = _layer([h3], [q4a, q4b], dega, degb, W4,
                   b4.reshape(1, -1), 128)

    Wc2p = jnp.zeros((D, D), _f32).at[:, :2].set(Wc2)
    bc2p = jnp.full((1, D), -1e30, _f32).at[0, :2].set(bc2)
    outp = _classifier(h4, Wc1, bc1.reshape(1, -1),
                       prelu_a.reshape(1, -1), Wc2p, bc2p)
    return outp[:, :2]
